# baseline (speedup 1.0000x reference)
import jax
import jax.numpy as jnp
from jax import lax
from jax.experimental import pallas as pl
from jax.experimental.pallas import tpu as pltpu
from jax.experimental.pallas import tpu_sc as plsc

IN_DIM = 256
HIDDEN = 1024
TABLE_ROWS = 1000
BATCH = 16384

NC, NS = 2, 16
NW = NC * NS
B_PER_W = BATCH // NW
CHUNK = 16
N_CHUNKS = B_PER_W // CHUNK
NBUF = 6
LOOKAHEAD = 3


def _mlp_body(emb_ref, w1_ref, b1_ref, w2_ref, b2_ref, out_ref):
    h = jnp.dot(emb_ref[...], w1_ref[...], preferred_element_type=jnp.float32)
    h = h + b1_ref[...]
    h = h * jax.nn.sigmoid(h)
    o = jnp.dot(h, w2_ref[...], preferred_element_type=jnp.float32)
    o = o + b2_ref[...]
    out_ref[...] = o * jax.nn.sigmoid(o)


def _sc_gather_body(table_hbm, idx_hbm, out_hbm, idx_v, rows_v,
                    g0, g1, g2, g3, g4, g5, w0, w1, w2, w3, w4, w5):
    gs = [g0, g1, g2, g3, g4, g5]
    ws = [w0, w1, w2, w3, w4, w5]
    wid = lax.axis_index("s") * NC + lax.axis_index("c")
    base = wid * B_PER_W
    pltpu.sync_copy(idx_hbm.at[wid], idx_v)
    g = [pltpu.async_copy(table_hbm.at[idx_v.at[b]], rows_v.at[b], gs[b])
         for b in range(NBUF)]
    w = [None] * NBUF
    for j in range(N_CHUNKS):
        b = j % NBUF
        g[b].wait()
        w[b] = pltpu.async_copy(rows_v.at[b],
                                out_hbm.at[pl.ds(base + j * CHUNK, CHUNK)],
                                ws[b])
        k = j + 1 + LOOKAHEAD
        if NBUF <= k < N_CHUNKS:
            bb = k % NBUF
            w[bb].wait()
            g[bb] = pltpu.async_copy(table_hbm.at[idx_v.at[k]],
                                     rows_v.at[bb], gs[bb])
    for b in range(NBUF):
        w[b].wait()


def kernel(diffusion_step, embedding, W1, b1, W2, b2):
    table = pl.pallas_call(
        _mlp_body,
        out_shape=jax.ShapeDtypeStruct((TABLE_ROWS, HIDDEN), jnp.float32),
    )(embedding, W1, b1.reshape(1, HIDDEN), W2, b2.reshape(1, HIDDEN))

    idx = diffusion_step.astype(jnp.int32).reshape(NW, N_CHUNKS, CHUNK)
    out = pl.kernel(
        _sc_gather_body,
        out_type=jax.ShapeDtypeStruct((BATCH, HIDDEN), jnp.float32),
        mesh=plsc.VectorSubcoreMesh(core_axis_name="c", subcore_axis_name="s"),
        scratch_types=(
            [pltpu.VMEM((N_CHUNKS, CHUNK), jnp.int32),
             pltpu.VMEM((NBUF, CHUNK, HIDDEN), jnp.float32)]
            + [pltpu.SemaphoreType.DMA] * (2 * NBUF)
        ),
    )(table, idx)
    return out
